# baseline (device time: 23698 ns/iter reference)
import jax
import jax.numpy as jnp
from jax import lax
from jax.experimental import pallas as pl
from jax.experimental.pallas import tpu as pltpu

N_DEV = 4


def kernel(q, k, v):
    s_per, d = q.shape
    half = s_per // 2
    scale = 1.0 / (d ** 0.5)

    def body(q_ref, k_ref, v_ref, o_ref, comm_ref, send_sems, recv_sems):
        my_pos = lax.axis_index("i")

        for h in (0, 1):
            comm_ref[0, h, pl.ds(0, half), :] = (
                k_ref[pl.ds(h * half, half), :].astype(jnp.bfloat16))
            comm_ref[0, h, pl.ds(half, half), :] = (
                v_ref[pl.ds(h * half, half), :].astype(jnp.bfloat16))

        barrier_sem = pltpu.get_barrier_semaphore()
        for off in (1, 2, 3):
            pl.semaphore_signal(
                barrier_sem, inc=1,
                device_id=((my_pos + off) % N_DEV,),
                device_id_type=pl.DeviceIdType.MESH,
            )
        pl.semaphore_wait(barrier_sem, 3)

        rdmas = {}
        for off in (1, 2, 3):
            dst_slot = N_DEV - off
            for h in (0, 1):
                rdma = pltpu.make_async_remote_copy(
                    src_ref=comm_ref.at[0, h],
                    dst_ref=comm_ref.at[dst_slot, h],
                    send_sem=send_sems.at[(off - 1) * 2 + h],
                    recv_sem=recv_sems.at[dst_slot * 2 + h],
                    device_id=((my_pos + off) % N_DEV,),
                    device_id_type=pl.DeviceIdType.MESH,
                )
                rdma.start()
                rdmas[(dst_slot, h)] = rdma

        q_bf = q_ref[:, :].astype(jnp.bfloat16)
        m = jnp.full((s_per, 1), -1e30, dtype=jnp.float32)
        l = jnp.zeros((s_per, 1), dtype=jnp.float32)
        acc = jnp.zeros((s_per, d), dtype=jnp.float32)

        def accumulate(slot, h, m, l, acc):
            k_h = comm_ref[slot, h, pl.ds(0, half), :]
            v_h = comm_ref[slot, h, pl.ds(half, half), :]
            s = lax.dot_general(
                q_bf, k_h,
                dimension_numbers=(((1,), (1,)), ((), ())),
                preferred_element_type=jnp.float32,
            ) * scale
            m_new = jnp.maximum(m, jnp.max(s, axis=1, keepdims=True))
            p = jnp.exp(s - m_new)
            alpha = jnp.exp(m - m_new)
            l = l * alpha + jnp.sum(p, axis=1, keepdims=True)
            acc = acc * alpha + lax.dot_general(
                p.astype(jnp.bfloat16), v_h,
                dimension_numbers=(((1,), (0,)), ((), ())),
                preferred_element_type=jnp.float32,
            )
            return m_new, l, acc

        for h in (0, 1):
            m, l, acc = accumulate(0, h, m, l, acc)
        for slot, h in ((3, 0), (1, 0), (3, 1), (1, 1), (2, 0), (2, 1)):
            rdmas[(slot, h)].wait_recv()
            m, l, acc = accumulate(slot, h, m, l, acc)

        for rdma in rdmas.values():
            rdma.wait_send()

        o_ref[:, :] = acc / l

    return pl.pallas_call(
        body,
        out_shape=jax.ShapeDtypeStruct((s_per, d), jnp.float32),
        in_specs=[
            pl.BlockSpec(memory_space=pltpu.VMEM),
            pl.BlockSpec(memory_space=pltpu.VMEM),
            pl.BlockSpec(memory_space=pltpu.VMEM),
        ],
        out_specs=pl.BlockSpec(memory_space=pltpu.VMEM),
        scratch_shapes=[
            pltpu.VMEM((N_DEV, 2, s_per, d), jnp.bfloat16),
            pltpu.SemaphoreType.DMA((6,)),
            pltpu.SemaphoreType.DMA((8,)),
        ],
        compiler_params=pltpu.CompilerParams(collective_id=0),
    )(q, k, v)
